# baseline (speedup 1.0000x reference)
import functools

import jax
import jax.numpy as jnp
from jax import lax
from jax.experimental import pallas as pl
from jax.experimental.pallas import tpu as pltpu
from jax.experimental.pallas import tpu_sc as plsc

_B, _L, _E = 4096, 200, 128
_H, _C = 256, 128
_NC, _NS = 2, 16
_NW = _NC * _NS
_RPW = _B // _NW
_HALF = _L // 2


def _sc_pool(idx_w, idx_b, idx_t, emb_w, emb_b, emb_t):
    mesh = plsc.VectorSubcoreMesh(core_axis_name="c", subcore_axis_name="s")
    out_t = [jax.ShapeDtypeStruct((_B, _E), jnp.float32) for _ in range(3)]
    scratch = [
        pltpu.VMEM((2 * _RPW, _HALF), jnp.int32),
        pltpu.VMEM((_L, _E), jnp.float32),
        pltpu.VMEM((_L, _E), jnp.float32),
        pltpu.VMEM((_RPW, _E), jnp.float32),
        pltpu.SemaphoreType.DMA,
        pltpu.SemaphoreType.DMA,
    ]

    @functools.partial(pl.kernel, mesh=mesh, out_type=out_t, scratch_types=scratch)
    def k(iw, ib, it, ew, eb, et, ow, ob, ot, idx_v, buf0, buf1, sums, sem0, sem1):
        wid = lax.axis_index("s") * _NC + lax.axis_index("c")
        base = wid * _RPW

        for idx_hbm, tab, out_hbm in ((iw, ew, ow), (ib, eb, ob), (it, et, ot)):
            pltpu.sync_copy(idx_hbm.at[pl.ds(2 * base, 2 * _RPW)], idx_v)

            def _gather(r, buf, sem, start, tab=tab):
                for j in range(2):
                    cp = pltpu.make_async_copy(
                        tab.at[idx_v.at[2 * r + j]],
                        buf.at[pl.ds(j * _HALF, _HALF)],
                        sem)
                    cp.start() if start else cp.wait()

            def _reduce_store(r, buf):
                def lbody(l, accs):
                    return tuple(accs[v] + buf[l, pl.ds(16 * v, 16)]
                                 for v in range(8))
                accs = lax.fori_loop(
                    0, _L, lbody,
                    tuple(jnp.zeros((16,), jnp.float32) for _ in range(8)),
                    unroll=8)
                for v in range(8):
                    sums[r, pl.ds(16 * v, 16)] = accs[v]

            _gather(0, buf0, sem0, start=True)

            def body(i, carry):
                r0 = 2 * i
                _gather(r0 + 1, buf1, sem1, start=True)
                _gather(r0, buf0, sem0, start=False)
                _reduce_store(r0, buf0)

                @pl.when(r0 + 2 < _RPW)
                def _():
                    _gather(r0 + 2, buf0, sem0, start=True)

                _gather(r0 + 1, buf1, sem1, start=False)
                _reduce_store(r0 + 1, buf1)
                return carry

            lax.fori_loop(0, _RPW // 2, body, 0)
            pltpu.sync_copy(sums, out_hbm.at[pl.ds(base, _RPW)])

    return k(idx_w, idx_b, idx_t, emb_w, emb_b, emb_t)


def _mlp(sw, sb, st, W1, b1, W2, b2):
    w1w = W1[:, 0:_E].T
    w1b = W1[:, _E:2 * _E].T
    w1t = W1[:, 2 * _E:3 * _E].T
    w2t = W2.T
    b1r = b1.reshape(1, _H)
    b2r = b2.reshape(1, _C)
    blk = 1024

    def body(swr, sbr, strr, w1wr, w1br, w1tr, b1r_, w2r, b2r_, outr):
        scale = jnp.float32(1.0 / _L)
        h = jnp.dot(swr[...] * scale, w1wr[...], preferred_element_type=jnp.float32)
        h = h + jnp.dot(sbr[...] * scale, w1br[...], preferred_element_type=jnp.float32)
        h = h + jnp.dot(strr[...] * scale, w1tr[...], preferred_element_type=jnp.float32)
        h = jnp.maximum(h + b1r_[...], 0.0)
        outr[...] = jnp.dot(h, w2r[...], preferred_element_type=jnp.float32) + b2r_[...]

    return pl.pallas_call(
        body,
        grid=(_B // blk,),
        in_specs=[
            pl.BlockSpec((blk, _E), lambda i: (i, 0)),
            pl.BlockSpec((blk, _E), lambda i: (i, 0)),
            pl.BlockSpec((blk, _E), lambda i: (i, 0)),
            pl.BlockSpec((_E, _H), lambda i: (0, 0)),
            pl.BlockSpec((_E, _H), lambda i: (0, 0)),
            pl.BlockSpec((_E, _H), lambda i: (0, 0)),
            pl.BlockSpec((1, _H), lambda i: (0, 0)),
            pl.BlockSpec((_H, _C), lambda i: (0, 0)),
            pl.BlockSpec((1, _C), lambda i: (0, 0)),
        ],
        out_specs=pl.BlockSpec((blk, _C), lambda i: (i, 0)),
        out_shape=jax.ShapeDtypeStruct((_B, _C), jnp.float32),
    )(sw, sb, st, w1w, w1b, w1t, b1r, w2t, b2r)


def kernel(inputs, bigram, trigram, emb_word, emb_bi, emb_tri, W1, b1, W2, b2):
    iw = inputs.astype(jnp.int32).reshape(2 * _B, _HALF)
    ib = bigram.astype(jnp.int32).reshape(2 * _B, _HALF)
    it = trigram.astype(jnp.int32).reshape(2 * _B, _HALF)
    sw, sb, st = _sc_pool(iw, ib, it, emb_word, emb_bi, emb_tri)
    return _mlp(sw, sb, st, W1, b1, W2, b2)


# speedup vs baseline: 1.2348x; 1.2348x over previous
import functools

import jax
import jax.numpy as jnp
from jax import lax
from jax.experimental import pallas as pl
from jax.experimental.pallas import tpu as pltpu
from jax.experimental.pallas import tpu_sc as plsc

_B, _L, _E = 4096, 200, 128
_H, _C = 256, 128
_NC, _NS = 2, 16
_NW = _NC * _NS
_RPW = _B // _NW
_HALF = _L // 2


def _sc_pool(idx_w, idx_b, idx_t, emb_w, emb_b, emb_t):
    mesh = plsc.VectorSubcoreMesh(core_axis_name="c", subcore_axis_name="s")
    out_t = [jax.ShapeDtypeStruct((_B, _E), jnp.float32) for _ in range(3)]
    scratch = [
        pltpu.VMEM((2 * _RPW, _HALF), jnp.int32),
        pltpu.VMEM((_HALF, _E), jnp.float32),
        pltpu.VMEM((_HALF, _E), jnp.float32),
        pltpu.VMEM((_HALF, _E), jnp.float32),
        pltpu.VMEM((_HALF, _E), jnp.float32),
        pltpu.VMEM((_RPW, _E), jnp.float32),
        pltpu.SemaphoreType.DMA,
        pltpu.SemaphoreType.DMA,
        pltpu.SemaphoreType.DMA,
        pltpu.SemaphoreType.DMA,
    ]

    @functools.partial(pl.kernel, mesh=mesh, out_type=out_t, scratch_types=scratch)
    def k(iw, ib, it, ew, eb, et, ow, ob, ot, idx_v,
          buf0, buf1, buf2, buf3, sums, sem0, sem1, sem2, sem3):
        wid = lax.axis_index("s") * _NC + lax.axis_index("c")
        base = wid * _RPW
        bufs = (buf0, buf1, buf2, buf3)
        sems = (sem0, sem1, sem2, sem3)
        _NU = 2 * _RPW

        for idx_hbm, tab, out_hbm in ((iw, ew, ow), (ib, eb, ob), (it, et, ot)):
            pltpu.sync_copy(idx_hbm.at[pl.ds(2 * base, 2 * _RPW)], idx_v)

            def _start(u, b, tab=tab):
                pltpu.make_async_copy(tab.at[idx_v.at[u]], bufs[b], sems[b]).start()

            def _wait(b, tab=tab):
                pltpu.make_async_copy(tab.at[idx_v.at[0]], bufs[b], sems[b]).wait()

            def _half_acc(b, accs):
                def lbody(l, a):
                    return tuple(a[v] + bufs[b][l, pl.ds(16 * v, 16)]
                                 for v in range(8))
                return lax.fori_loop(0, _HALF, lbody, accs, unroll=4)

            def _store(r, accs):
                for v in range(8):
                    sums[r, pl.ds(16 * v, 16)] = accs[v]

            _zeros = tuple(jnp.zeros((16,), jnp.float32) for _ in range(8))
            for b in range(4):
                _start(b, b)

            def body(i, carry):
                u = 4 * i
                accs = None
                for b in range(4):
                    _wait(b)
                    accs = _half_acc(b, _zeros if b % 2 == 0 else accs)
                    if b % 2 == 1:
                        _store(2 * i + b // 2, accs)

                    @pl.when(u + 4 + b < _NU)
                    def _(u=u, b=b):
                        _start(u + 4 + b, b)
                return carry

            lax.fori_loop(0, _RPW // 2, body, 0)
            pltpu.sync_copy(sums, out_hbm.at[pl.ds(base, _RPW)])

    return k(idx_w, idx_b, idx_t, emb_w, emb_b, emb_t)


def _mlp(sw, sb, st, W1, b1, W2, b2):
    w1w = W1[:, 0:_E].T
    w1b = W1[:, _E:2 * _E].T
    w1t = W1[:, 2 * _E:3 * _E].T
    w2t = W2.T
    b1r = b1.reshape(1, _H)
    b2r = b2.reshape(1, _C)
    blk = 1024

    def body(swr, sbr, strr, w1wr, w1br, w1tr, b1r_, w2r, b2r_, outr):
        scale = jnp.float32(1.0 / _L)
        h = jnp.dot(swr[...] * scale, w1wr[...], preferred_element_type=jnp.float32)
        h = h + jnp.dot(sbr[...] * scale, w1br[...], preferred_element_type=jnp.float32)
        h = h + jnp.dot(strr[...] * scale, w1tr[...], preferred_element_type=jnp.float32)
        h = jnp.maximum(h + b1r_[...], 0.0)
        outr[...] = jnp.dot(h, w2r[...], preferred_element_type=jnp.float32) + b2r_[...]

    return pl.pallas_call(
        body,
        grid=(_B // blk,),
        in_specs=[
            pl.BlockSpec((blk, _E), lambda i: (i, 0)),
            pl.BlockSpec((blk, _E), lambda i: (i, 0)),
            pl.BlockSpec((blk, _E), lambda i: (i, 0)),
            pl.BlockSpec((_E, _H), lambda i: (0, 0)),
            pl.BlockSpec((_E, _H), lambda i: (0, 0)),
            pl.BlockSpec((_E, _H), lambda i: (0, 0)),
            pl.BlockSpec((1, _H), lambda i: (0, 0)),
            pl.BlockSpec((_H, _C), lambda i: (0, 0)),
            pl.BlockSpec((1, _C), lambda i: (0, 0)),
        ],
        out_specs=pl.BlockSpec((blk, _C), lambda i: (i, 0)),
        out_shape=jax.ShapeDtypeStruct((_B, _C), jnp.float32),
    )(sw, sb, st, w1w, w1b, w1t, b1r, w2t, b2r)


def kernel(inputs, bigram, trigram, emb_word, emb_bi, emb_tri, W1, b1, W2, b2):
    iw = inputs.astype(jnp.int32).reshape(2 * _B, _HALF)
    ib = bigram.astype(jnp.int32).reshape(2 * _B, _HALF)
    it = trigram.astype(jnp.int32).reshape(2 * _B, _HALF)
    sw, sb, st = _sc_pool(iw, ib, it, emb_word, emb_bi, emb_tri)
    return _mlp(sw, sb, st, W1, b1, W2, b2)


# speedup vs baseline: 1.2434x; 1.0070x over previous
import functools

import jax
import jax.numpy as jnp
from jax import lax
from jax.experimental import pallas as pl
from jax.experimental.pallas import tpu as pltpu
from jax.experimental.pallas import tpu_sc as plsc

_B, _L, _E = 4096, 200, 128
_H, _C = 256, 128
_NC, _NS = 2, 16
_NW = _NC * _NS
_RPW = _B // _NW
_HALF = _L // 2
_QTR = _L // 4
_RING = 8


def _sc_pool(idx_w, idx_b, idx_t, emb_w, emb_b, emb_t):
    mesh = plsc.VectorSubcoreMesh(core_axis_name="c", subcore_axis_name="s")
    out_t = [jax.ShapeDtypeStruct((_B, _E), jnp.float32) for _ in range(3)]
    _UPR = _L // _QTR
    _NU = _UPR * _RPW
    scratch = (
        [pltpu.VMEM((_NU, _QTR), jnp.int32)]
        + [pltpu.VMEM((_QTR, _E), jnp.float32) for _ in range(_RING)]
        + [pltpu.VMEM((16, _E), jnp.float32)]
        + [pltpu.SemaphoreType.DMA for _ in range(_RING)]
    )

    @functools.partial(pl.kernel, mesh=mesh, out_type=out_t, scratch_types=scratch)
    def k(iw, ib, it, ew, eb, et, ow, ob, ot, idx_v, *rest):
        bufs = rest[:_RING]
        sums = rest[_RING]
        sems = rest[_RING + 1:]
        wid = lax.axis_index("s") * _NC + lax.axis_index("c")
        base = wid * _RPW

        for idx_hbm, tab, out_hbm in ((iw, ew, ow), (ib, eb, ob), (it, et, ot)):
            pltpu.sync_copy(idx_hbm.at[pl.ds(_UPR * base, _NU)], idx_v)

            def _start(u, b, tab=tab):
                pltpu.make_async_copy(tab.at[idx_v.at[u]], bufs[b], sems[b]).start()

            def _wait(b, tab=tab):
                pltpu.make_async_copy(tab.at[idx_v.at[0]], bufs[b], sems[b]).wait()

            def _unit_acc(b, accs):
                def lbody(l, a):
                    return tuple(a[v] + bufs[b][l, pl.ds(16 * v, 16)]
                                 for v in range(8))
                return lax.fori_loop(0, _QTR, lbody, accs, unroll=5)

            def _store(slot, accs):
                for v in range(8):
                    sums[slot, pl.ds(16 * v, 16)] = accs[v]

            _zeros = tuple(jnp.zeros((16,), jnp.float32) for _ in range(8))
            for b in range(_RING):
                _start(b, b)

            def body(i, carry):
                u = _RING * i
                accs = None
                for b in range(_RING):
                    _wait(b)
                    accs = _unit_acc(b, _zeros if b % _UPR == 0 else accs)
                    if b % _UPR == _UPR - 1:
                        _store(lax.rem(2 * i + b // _UPR, 16), accs)

                    @pl.when(u + _RING + b < _NU)
                    def _(u=u, b=b):
                        _start(u + _RING + b, b)

                @pl.when(lax.rem(i, 8) == 7)
                def _(out_hbm=out_hbm):
                    off = pl.multiple_of(base + 2 * i - 14, 16)
                    pltpu.sync_copy(sums, out_hbm.at[pl.ds(off, 16)])
                return carry

            lax.fori_loop(0, _NU // _RING, body, 0)

    return k(idx_w, idx_b, idx_t, emb_w, emb_b, emb_t)


def _mlp(sw, sb, st, W1, b1, W2, b2):
    w1w = W1[:, 0:_E].T
    w1b = W1[:, _E:2 * _E].T
    w1t = W1[:, 2 * _E:3 * _E].T
    w2t = W2.T
    b1r = b1.reshape(1, _H)
    b2r = b2.reshape(1, _C)
    blk = 1024

    def body(swr, sbr, strr, w1wr, w1br, w1tr, b1r_, w2r, b2r_, outr):
        scale = jnp.float32(1.0 / _L)
        h = jnp.dot(swr[...] * scale, w1wr[...], preferred_element_type=jnp.float32)
        h = h + jnp.dot(sbr[...] * scale, w1br[...], preferred_element_type=jnp.float32)
        h = h + jnp.dot(strr[...] * scale, w1tr[...], preferred_element_type=jnp.float32)
        h = jnp.maximum(h + b1r_[...], 0.0)
        outr[...] = jnp.dot(h, w2r[...], preferred_element_type=jnp.float32) + b2r_[...]

    return pl.pallas_call(
        body,
        grid=(_B // blk,),
        in_specs=[
            pl.BlockSpec((blk, _E), lambda i: (i, 0)),
            pl.BlockSpec((blk, _E), lambda i: (i, 0)),
            pl.BlockSpec((blk, _E), lambda i: (i, 0)),
            pl.BlockSpec((_E, _H), lambda i: (0, 0)),
            pl.BlockSpec((_E, _H), lambda i: (0, 0)),
            pl.BlockSpec((_E, _H), lambda i: (0, 0)),
            pl.BlockSpec((1, _H), lambda i: (0, 0)),
            pl.BlockSpec((_H, _C), lambda i: (0, 0)),
            pl.BlockSpec((1, _C), lambda i: (0, 0)),
        ],
        out_specs=pl.BlockSpec((blk, _C), lambda i: (i, 0)),
        out_shape=jax.ShapeDtypeStruct((_B, _C), jnp.float32),
    )(sw, sb, st, w1w, w1b, w1t, b1r, w2t, b2r)


def kernel(inputs, bigram, trigram, emb_word, emb_bi, emb_tri, W1, b1, W2, b2):
    iw = inputs.astype(jnp.int32).reshape(4 * _B, _QTR)
    ib = bigram.astype(jnp.int32).reshape(4 * _B, _QTR)
    it = trigram.astype(jnp.int32).reshape(4 * _B, _QTR)
    sw, sb, st = _sc_pool(iw, ib, it, emb_word, emb_bi, emb_tri)
    return _mlp(sw, sb, st, W1, b1, W2, b2)
